# baseline (device time: 86237 ns/iter reference)
import jax
import jax.numpy as jnp
from jax import lax
from jax.experimental import pallas as pl
from jax.experimental.pallas import tpu as pltpu

T = 1024
D = 1024
V_LOCAL = 8192


def kernel(ids, E):
    my_y = lax.axis_index("y")
    offset = my_y * V_LOCAL
    local = ids - offset
    mask = (local >= 0) & (local < V_LOCAL)
    safe = jnp.where(mask, local, 0)
    partial = jnp.where(mask[:, None], jnp.take(E, safe, axis=0), 0.0)

    def body(partial_ref, out_ref, comm_ref, send_sem, recv_sem):
        my_x = lax.axis_index("x")
        my_y = lax.axis_index("y")
        nbr = (my_x, 1 - my_y)

        barrier_sem = pltpu.get_barrier_semaphore()
        pl.semaphore_signal(
            barrier_sem, inc=1, device_id=nbr,
            device_id_type=pl.DeviceIdType.MESH,
        )
        pl.semaphore_wait(barrier_sem, 1)

        rdma = pltpu.make_async_remote_copy(
            src_ref=partial_ref,
            dst_ref=comm_ref,
            send_sem=send_sem,
            recv_sem=recv_sem,
            device_id=nbr,
            device_id_type=pl.DeviceIdType.MESH,
        )
        rdma.start()
        rdma.wait()

        out_ref[...] = partial_ref[...] + comm_ref[...]

    return pl.pallas_call(
        body,
        out_shape=jax.ShapeDtypeStruct((T, D), jnp.float32),
        in_specs=[pl.BlockSpec(memory_space=pltpu.VMEM)],
        out_specs=pl.BlockSpec(memory_space=pltpu.VMEM),
        scratch_shapes=[
            pltpu.VMEM((T, D), jnp.float32),
            pltpu.SemaphoreType.DMA,
            pltpu.SemaphoreType.DMA,
        ],
        compiler_params=pltpu.CompilerParams(collective_id=0),
    )(partial)


# device time: 47302 ns/iter; 1.8231x vs baseline; 1.8231x over previous
import jax
import jax.numpy as jnp
from jax import lax
from jax.experimental import pallas as pl
from jax.experimental.pallas import tpu as pltpu

T = 1024
D = 1024
V_LOCAL = 8192
HALF = T // 2
K = 8
CHUNK = HALF // K


def kernel(ids, E):
    my_x = lax.axis_index("x")
    my_y = lax.axis_index("y")

    ids_half = lax.dynamic_slice(ids, (my_x * HALF,), (HALF,))
    local = ids_half - my_y * V_LOCAL
    mask = (local >= 0) & (local < V_LOCAL)
    partial = jnp.where(
        mask[:, None], jnp.take(E, jnp.where(mask, local, 0), axis=0), 0.0
    )

    def body(partial_ref, out_ref, comm_ref, y_send, y_recv, x_send, x_recv):
        my_x = lax.axis_index("x")
        my_y = lax.axis_index("y")
        y_nbr = (my_x, 1 - my_y)
        x_nbr = (1 - my_x, my_y)

        barrier_sem = pltpu.get_barrier_semaphore()
        for nbr in (y_nbr, x_nbr):
            pl.semaphore_signal(
                barrier_sem, inc=1, device_id=nbr,
                device_id_type=pl.DeviceIdType.MESH,
            )
        pl.semaphore_wait(barrier_sem, 2)

        y_rdmas = []
        for k in range(K):
            sl = pl.ds(k * CHUNK, CHUNK)
            r = pltpu.make_async_remote_copy(
                src_ref=partial_ref.at[sl, :],
                dst_ref=comm_ref.at[sl, :],
                send_sem=y_send.at[k],
                recv_sem=y_recv.at[k],
                device_id=y_nbr,
                device_id_type=pl.DeviceIdType.MESH,
            )
            r.start()
            y_rdmas.append(r)

        my_base = my_x * HALF
        x_rdmas = []
        for k in range(K):
            sl = pl.ds(k * CHUNK, CHUNK)
            out_sl = pl.ds(my_base + k * CHUNK, CHUNK)
            y_rdmas[k].wait_recv()
            out_ref[out_sl, :] = partial_ref[sl, :] + comm_ref[sl, :]
            r = pltpu.make_async_remote_copy(
                src_ref=out_ref.at[out_sl, :],
                dst_ref=out_ref.at[out_sl, :],
                send_sem=x_send.at[k],
                recv_sem=x_recv.at[k],
                device_id=x_nbr,
                device_id_type=pl.DeviceIdType.MESH,
            )
            r.start()
            x_rdmas.append(r)

        for k in range(K):
            y_rdmas[k].wait_send()
            x_rdmas[k].wait_send()
            x_rdmas[k].wait_recv()

    return pl.pallas_call(
        body,
        out_shape=jax.ShapeDtypeStruct((T, D), jnp.float32),
        in_specs=[pl.BlockSpec(memory_space=pltpu.VMEM)],
        out_specs=pl.BlockSpec(memory_space=pltpu.VMEM),
        scratch_shapes=[
            pltpu.VMEM((HALF, D), jnp.float32),
            pltpu.SemaphoreType.DMA((K,)),
            pltpu.SemaphoreType.DMA((K,)),
            pltpu.SemaphoreType.DMA((K,)),
            pltpu.SemaphoreType.DMA((K,)),
        ],
        compiler_params=pltpu.CompilerParams(collective_id=0),
    )(partial)


# device time: 45393 ns/iter; 1.8998x vs baseline; 1.0421x over previous
import jax
import jax.numpy as jnp
from jax import lax
from jax.experimental import pallas as pl
from jax.experimental.pallas import tpu as pltpu

T = 1024
D = 1024
V_LOCAL = 8192
HALF = T // 2
K = 8
CHUNK = HALF // K


def kernel(ids, E):
    my_x = lax.axis_index("x")
    my_y = lax.axis_index("y")

    ids_half = lax.dynamic_slice(ids, (my_x * HALF,), (HALF,))
    local = ids_half - my_y * V_LOCAL
    mask = (local >= 0) & (local < V_LOCAL)
    safe = jnp.clip(local, 0, V_LOCAL - 1).astype(jnp.int32)
    maskf = mask.astype(jnp.float32)[:, None]

    def body(safe_ref, maskf_ref, e_ref, out_ref,
             gather_ref, comm_ref, g_sems, y_send, y_recv, x_send, x_recv):
        my_x = lax.axis_index("x")
        my_y = lax.axis_index("y")
        y_nbr = (my_x, 1 - my_y)
        x_nbr = (1 - my_x, my_y)

        barrier_sem = pltpu.get_barrier_semaphore()
        for nbr in (y_nbr, x_nbr):
            pl.semaphore_signal(
                barrier_sem, inc=1, device_id=nbr,
                device_id_type=pl.DeviceIdType.MESH,
            )
        pl.semaphore_wait(barrier_sem, 2)

        def issue_row(i, _):
            row = safe_ref[i]
            pltpu.make_async_copy(
                e_ref.at[pl.ds(row, 1), :],
                gather_ref.at[pl.ds(i, 1), :],
                g_sems.at[i // CHUNK],
            ).start()
            return 0

        lax.fori_loop(0, HALF, issue_row, 0, unroll=4)

        def wait_gather_chunk(k):
            def w(i, _):
                pltpu.make_async_copy(
                    e_ref.at[pl.ds(0, 1), :],
                    gather_ref.at[pl.ds(0, 1), :],
                    g_sems.at[k],
                ).wait()
                return 0
            lax.fori_loop(0, CHUNK, w, 0, unroll=4)

        y_rdmas = []
        for k in range(K):
            sl = pl.ds(k * CHUNK, CHUNK)
            wait_gather_chunk(k)
            gather_ref[sl, :] = gather_ref[sl, :] * maskf_ref[sl, :]
            r = pltpu.make_async_remote_copy(
                src_ref=gather_ref.at[sl, :],
                dst_ref=comm_ref.at[sl, :],
                send_sem=y_send.at[k],
                recv_sem=y_recv.at[k],
                device_id=y_nbr,
                device_id_type=pl.DeviceIdType.MESH,
            )
            r.start()
            y_rdmas.append(r)

        my_base = my_x * HALF
        x_rdmas = []
        for k in range(K):
            sl = pl.ds(k * CHUNK, CHUNK)
            out_sl = pl.ds(my_base + k * CHUNK, CHUNK)
            y_rdmas[k].wait_recv()
            out_ref[out_sl, :] = gather_ref[sl, :] + comm_ref[sl, :]
            r = pltpu.make_async_remote_copy(
                src_ref=out_ref.at[out_sl, :],
                dst_ref=out_ref.at[out_sl, :],
                send_sem=x_send.at[k],
                recv_sem=x_recv.at[k],
                device_id=x_nbr,
                device_id_type=pl.DeviceIdType.MESH,
            )
            r.start()
            x_rdmas.append(r)

        for k in range(K):
            y_rdmas[k].wait_send()
            x_rdmas[k].wait_send()
            x_rdmas[k].wait_recv()

    return pl.pallas_call(
        body,
        out_shape=jax.ShapeDtypeStruct((T, D), jnp.float32),
        in_specs=[
            pl.BlockSpec(memory_space=pltpu.SMEM),
            pl.BlockSpec(memory_space=pltpu.VMEM),
            pl.BlockSpec(memory_space=pl.ANY),
        ],
        out_specs=pl.BlockSpec(memory_space=pltpu.VMEM),
        scratch_shapes=[
            pltpu.VMEM((HALF, D), jnp.float32),
            pltpu.VMEM((HALF, D), jnp.float32),
            pltpu.SemaphoreType.DMA((K,)),
            pltpu.SemaphoreType.DMA((K,)),
            pltpu.SemaphoreType.DMA((K,)),
            pltpu.SemaphoreType.DMA((K,)),
            pltpu.SemaphoreType.DMA((K,)),
        ],
        compiler_params=pltpu.CompilerParams(collective_id=0),
    )(safe, maskf, E)


# device time: 37823 ns/iter; 2.2800x vs baseline; 1.2001x over previous
import jax
import jax.numpy as jnp
from jax import lax
from jax.experimental import pallas as pl
from jax.experimental.pallas import tpu as pltpu

T = 1024
D = 1024
V_LOCAL = 8192
HALF = T // 2
K = 8
CHUNK = HALF // K


def kernel(ids, E):
    my_x = lax.axis_index("x")
    my_y = lax.axis_index("y")

    ids_half = lax.dynamic_slice(ids, (my_x * HALF,), (HALF,))
    local = ids_half - my_y * V_LOCAL
    mask = (local >= 0) & (local < V_LOCAL)
    safe = jnp.clip(local, 0, V_LOCAL - 1).astype(jnp.int32)
    maskf = mask.astype(jnp.float32)[:, None]

    def body(safe_ref, maskf_ref, e_ref, out_ref,
             gather_ref, comm_ref, g_sems, y_send, y_recv, x_send, x_recv):
        my_x = lax.axis_index("x")
        my_y = lax.axis_index("y")
        y_nbr = (my_x, 1 - my_y)
        x_nbr = (1 - my_x, my_y)
        my_base = my_x * HALF

        barrier_sem = pltpu.get_barrier_semaphore()
        for nbr in (y_nbr, x_nbr):
            pl.semaphore_signal(
                barrier_sem, inc=1, device_id=nbr,
                device_id_type=pl.DeviceIdType.MESH,
            )
        pl.semaphore_wait(barrier_sem, 2)

        def issue_gather(k):
            base = k * CHUNK

            def issue_row(i, _):
                pltpu.make_async_copy(
                    e_ref.at[pl.ds(safe_ref[base + i], 1), :],
                    gather_ref.at[pl.ds(base + i, 1), :],
                    g_sems.at[k],
                ).start()
                return 0

            lax.fori_loop(0, CHUNK, issue_row, 0, unroll=8)

        def wait_gather(k):
            def w(i, _):
                pltpu.make_async_copy(
                    e_ref.at[pl.ds(0, 1), :],
                    gather_ref.at[pl.ds(0, 1), :],
                    g_sems.at[k],
                ).wait()
                return 0

            lax.fori_loop(0, CHUNK, w, 0, unroll=8)

        y_rdmas = []
        x_rdmas = []

        def process(k):
            sl = pl.ds(k * CHUNK, CHUNK)
            out_sl = pl.ds(my_base + k * CHUNK, CHUNK)
            y_rdmas[k].wait_recv()
            out_ref[out_sl, :] = gather_ref[sl, :] + comm_ref[sl, :]
            r = pltpu.make_async_remote_copy(
                src_ref=out_ref.at[out_sl, :],
                dst_ref=out_ref.at[out_sl, :],
                send_sem=x_send.at[k],
                recv_sem=x_recv.at[k],
                device_id=x_nbr,
                device_id_type=pl.DeviceIdType.MESH,
            )
            r.start()
            x_rdmas.append(r)

        issue_gather(0)
        for k in range(K):
            if k + 1 < K:
                issue_gather(k + 1)
            sl = pl.ds(k * CHUNK, CHUNK)
            wait_gather(k)
            gather_ref[sl, :] = jnp.where(
                maskf_ref[sl, :] != 0.0, gather_ref[sl, :], 0.0
            )
            r = pltpu.make_async_remote_copy(
                src_ref=gather_ref.at[sl, :],
                dst_ref=comm_ref.at[sl, :],
                send_sem=y_send.at[k],
                recv_sem=y_recv.at[k],
                device_id=y_nbr,
                device_id_type=pl.DeviceIdType.MESH,
            )
            r.start()
            y_rdmas.append(r)
            if k >= 1:
                process(k - 1)
        process(K - 1)

        for k in range(K):
            y_rdmas[k].wait_send()
            x_rdmas[k].wait_send()
            x_rdmas[k].wait_recv()

    return pl.pallas_call(
        body,
        out_shape=jax.ShapeDtypeStruct((T, D), jnp.float32),
        in_specs=[
            pl.BlockSpec(memory_space=pltpu.SMEM),
            pl.BlockSpec(memory_space=pltpu.VMEM),
            pl.BlockSpec(memory_space=pl.ANY),
        ],
        out_specs=pl.BlockSpec(memory_space=pltpu.VMEM),
        scratch_shapes=[
            pltpu.VMEM((HALF, D), jnp.float32),
            pltpu.VMEM((HALF, D), jnp.float32),
            pltpu.SemaphoreType.DMA((K,)),
            pltpu.SemaphoreType.DMA((K,)),
            pltpu.SemaphoreType.DMA((K,)),
            pltpu.SemaphoreType.DMA((K,)),
            pltpu.SemaphoreType.DMA((K,)),
        ],
        compiler_params=pltpu.CompilerParams(collective_id=0),
    )(safe, maskf, E)
